# baseline (speedup 1.0000x reference)
import jax
import jax.numpy as jnp
from jax.experimental import pallas as pl


def _relu_bn_kernel(o_ref, mean_ref, scale_ref, beta_ref, out_ref):
    out_ref[...] = jnp.maximum(
        (o_ref[...] - mean_ref[...]) * scale_ref[...] + beta_ref[...], 0.0
    )


def kernel(x, edge_index, W, b, gamma, beta):
    N = x.shape[0]
    loop = jnp.arange(N, dtype=edge_index.dtype)
    src = jnp.concatenate([edge_index[0], loop])
    dst = jnp.concatenate([edge_index[1], loop])
    deg = jnp.zeros((N,), dtype=x.dtype).at[dst].add(1.0)
    deg_inv_sqrt = jnp.where(deg > 0, jax.lax.rsqrt(deg), 0.0)
    norm = deg_inv_sqrt[src] * deg_inv_sqrt[dst]
    xw = x @ W
    msgs = jnp.take(xw, src, axis=0) * norm[:, None]
    out = jnp.zeros((N, xw.shape[1]), dtype=x.dtype).at[dst].add(msgs)
    out = out + b
    mean = jnp.mean(out, axis=0)
    var = jnp.mean((out - mean) ** 2, axis=0)
    scale = jax.lax.rsqrt(var + 1e-5) * gamma
    return pl.pallas_call(
        _relu_bn_kernel,
        out_shape=jax.ShapeDtypeStruct(out.shape, out.dtype),
    )(out, mean[None, :], scale[None, :], beta[None, :])


# speedup vs baseline: 14.6471x; 14.6471x over previous
import functools

import jax
import jax.numpy as jnp
from jax import lax
from jax.experimental import pallas as pl
from jax.experimental.pallas import tpu as pltpu
from jax.experimental.pallas import tpu_sc as plsc

N = 10000
D = 128
E = 320000
NC, NS = 2, 16
NW = NC * NS
CHUNK = 128
EPT = 10112
NCHUNK = EPT // CHUNK
EPAD = NW * EPT
NPAD = 10112
ROWS_PT = NPAD // NS
HROWS = NPAD // 16
TRASH = 10000
HB = 8
EPS = 1e-5

_mesh = plsc.VectorSubcoreMesh(core_axis_name="c", subcore_axis_name="s")


@functools.partial(
    pl.kernel,
    out_type=jax.ShapeDtypeStruct((NW, HROWS, 128), jnp.float32),
    mesh=_mesh,
    scratch_types=[
        pltpu.VMEM((CHUNK,), jnp.int32),
        pltpu.VMEM((HROWS, 128), jnp.float32),
    ],
    compiler_params=pltpu.CompilerParams(needs_layout_passes=False),
)
def _deg_sc(dst_hbm, histout_hbm, idx_v, hist_v):
    c = lax.axis_index("c")
    s = lax.axis_index("s")
    w = c * NS + s
    base = w * EPT

    @pl.loop(0, HROWS)
    def _(i):
        for j in range(8):
            hist_v[i, pl.ds(j * 16, 16)] = jnp.zeros((16,), jnp.float32)

    lane = lax.iota(jnp.int32, 16)
    bank = lane & (HB - 1)
    lo = lane < 8
    ones = jnp.ones((16,), jnp.float32)

    @pl.loop(0, NCHUNK)
    def _(i):
        pltpu.sync_copy(dst_hbm.at[pl.ds(base + i * CHUNK, CHUNK)], idx_v)
        for j in range(CHUNK // 16):
            idx16 = idx_v[pl.ds(j * 16, 16)]
            row = idx16 >> 4
            col = (idx16 & 15) * HB + bank
            plsc.addupdate_scatter(hist_v, [row, col], ones, mask=lo)
            plsc.addupdate_scatter(hist_v, [row, col], ones, mask=~lo)

    pltpu.sync_copy(hist_v, histout_hbm.at[w])


@functools.partial(
    pl.kernel,
    out_type=jax.ShapeDtypeStruct((NC, NPAD, D), jnp.float32),
    mesh=_mesh,
    scratch_types=[
        pltpu.VMEM((CHUNK,), jnp.int32),
        pltpu.VMEM((CHUNK,), jnp.int32),
        pltpu.VMEM((CHUNK, D), jnp.float32),
        pltpu.VMEM_SHARED((NPAD, D), jnp.float32),
        pltpu.SemaphoreType.DMA,
    ],
)
def _scatter_sc(y_hbm, src_hbm, dst_hbm, zout_hbm, sidx_v, didx_v, rows_v,
                acc_sh, sem):
    c = lax.axis_index("c")
    s = lax.axis_index("s")
    base = (c * NS + s) * EPT
    row0 = s * ROWS_PT

    @pl.loop(0, CHUNK)
    def _(i):
        for j in range(D // 16):
            rows_v[i, pl.ds(j * 16, 16)] = jnp.zeros((16,), jnp.float32)

    for k in range(ROWS_PT // CHUNK):
        pltpu.sync_copy(rows_v, acc_sh.at[pl.ds(row0 + k * CHUNK, CHUNK)])
    _rem = ROWS_PT % CHUNK
    pltpu.sync_copy(rows_v.at[pl.ds(0, _rem)],
                    acc_sh.at[pl.ds(row0 + (ROWS_PT // CHUNK) * CHUNK, _rem)])
    plsc.subcore_barrier()

    @pl.loop(0, NCHUNK)
    def _(i):
        off = base + i * CHUNK
        pltpu.sync_copy(src_hbm.at[pl.ds(off, CHUNK)], sidx_v)
        pltpu.sync_copy(dst_hbm.at[pl.ds(off, CHUNK)], didx_v)
        pltpu.async_copy(y_hbm.at[sidx_v], rows_v, sem).wait()
        pltpu.sync_copy(rows_v, acc_sh.at[didx_v], add=True)

    plsc.subcore_barrier()
    pltpu.sync_copy(acc_sh.at[pl.ds(row0, ROWS_PT)],
                    zout_hbm.at[c, pl.ds(row0, ROWS_PT)])


def _mm_body(x_ref, w_ref, o_ref):
    o_ref[...] = jnp.dot(x_ref[...], w_ref[...],
                         preferred_element_type=jnp.float32)


_mm = pl.pallas_call(
    _mm_body, out_shape=jax.ShapeDtypeStruct((N, D), jnp.float32))


def _degred_body(histp_ref, o_ref):
    h = histp_ref[0]
    for w in range(1, NW):
        h = h + histp_ref[w]
    li = lax.broadcasted_iota(jnp.int32, (128, 16), 0)
    lj = lax.broadcasted_iota(jnp.int32, (128, 16), 1)
    bsel = (li // HB == lj).astype(jnp.float32)
    deg16 = jnp.dot(h, bsel, preferred_element_type=jnp.float32) + 1.0
    o_ref[...] = lax.rsqrt(deg16)


_degred = pl.pallas_call(
    _degred_body, out_shape=jax.ShapeDtypeStruct((HROWS, 16), jnp.float32))


def _scale_body(xw_ref, dinv_ref, y_ref):
    y_ref[...] = xw_ref[...] * dinv_ref[...]


_scale = pl.pallas_call(
    _scale_body, out_shape=jax.ShapeDtypeStruct((N, D), jnp.float32))


def _final_body(z_ref, y_ref, dinv_ref, b_ref, g_ref, be_ref, o_ref):
    z = z_ref[0, :N, :] + z_ref[1, :N, :] + y_ref[...]
    pre = z * dinv_ref[...] + b_ref[...]
    mean = jnp.mean(pre, axis=0, keepdims=True)
    var = jnp.mean((pre - mean) ** 2, axis=0, keepdims=True)
    o_ref[...] = jnp.maximum(
        (pre - mean) * lax.rsqrt(var + EPS) * g_ref[...] + be_ref[...], 0.0)


_final = pl.pallas_call(
    _final_body, out_shape=jax.ShapeDtypeStruct((N, D), jnp.float32))


def kernel(x, edge_index, W, b, gamma, beta):
    src = edge_index[0].astype(jnp.int32)
    dst = edge_index[1].astype(jnp.int32)
    pad = EPAD - E
    src_p = jnp.concatenate([src, jnp.zeros((pad,), jnp.int32)])
    dst_p = jnp.concatenate([dst, jnp.full((pad,), TRASH, jnp.int32)])

    xw = _mm(x, W)
    histp = _deg_sc(dst_p)
    dinv = _degred(histp).reshape(NPAD, 1)[:N]
    y = _scale(xw, dinv)
    zp = _scatter_sc(y, src_p, dst_p)
    return _final(zp, y, dinv, b.reshape(1, D), gamma.reshape(1, D),
                  beta.reshape(1, D))
